# Initial kernel scaffold; baseline (speedup 1.0000x reference)
import jax
import jax.numpy as jnp
from jax.experimental import pallas as pl


def kernel(x_patient, x_visit, edge_index_pv, edge_index_vp, l1_pv_Wsrc, l1_pv_Wdst, l1_pv_asrc, l1_pv_adst, l1_pv_b, l1_vp_Wsrc, l1_vp_Wdst, l1_vp_asrc, l1_vp_adst, l1_vp_b, l2_pv_Wsrc, l2_pv_Wdst, l2_pv_asrc, l2_pv_adst, l2_pv_b, l2_vp_Wsrc, l2_vp_Wdst, l2_vp_asrc, l2_vp_adst, l2_vp_b, head_p_W, head_p_b, head_v_W, head_v_b):
    raise NotImplementedError("write your pallas kernel here")


# speedup vs baseline: 24.9390x; 24.9390x over previous
import functools

import jax
import jax.numpy as jnp
from jax import lax
from jax.experimental import pallas as pl
from jax.experimental.pallas import tpu as pltpu
from jax.experimental.pallas import tpu_sc as plsc

N = 10000
E = 160000
D = 128
H = 64
OUT = 32
ROW = 80
SUB = 80
NSUB = 5
CHUNK = SUB * NSUB
NW = 16
EPW = E // NW
NCHUNK = EPW // CHUNK
RPW = N // NW

_f32 = jnp.float32


def _dot(a, b):
  return jnp.dot(a, b, preferred_element_type=_f32)


def _dense_body(xa_ref, xb_ref,
                ws_ab_ref, as_ab_ref, wd_ab_ref, ad_ab_ref,
                ws_ba_ref, as_ba_ref, wd_ba_ref, ad_ba_ref,
                hs_ab_ref, ss_ab_ref, sd_ab_ref,
                hs_ba_ref, ss_ba_ref, sd_ba_ref):
  xa = xa_ref[...]
  xb = xb_ref[...]
  hs_ab = _dot(xa, ws_ab_ref[...])
  hs_ab_ref[...] = hs_ab
  ss_ab_ref[...] = _dot(hs_ab, as_ab_ref[...])
  sd_ab_ref[...] = _dot(_dot(xb, wd_ab_ref[...]), ad_ab_ref[...])
  hs_ba = _dot(xb, ws_ba_ref[...])
  hs_ba_ref[...] = hs_ba
  ss_ba_ref[...] = _dot(hs_ba, as_ba_ref[...])
  sd_ba_ref[...] = _dot(_dot(xa, wd_ba_ref[...]), ad_ba_ref[...])


def _dense_pair(xa, xb, ws_ab, aas_ab, wd_ab, ad_ab, ws_ba, aas_ba, wd_ba, ad_ba):
  out_shape = (
      jax.ShapeDtypeStruct((N, H), _f32),
      jax.ShapeDtypeStruct((N, 1), _f32),
      jax.ShapeDtypeStruct((N, 1), _f32),
      jax.ShapeDtypeStruct((N, H), _f32),
      jax.ShapeDtypeStruct((N, 1), _f32),
      jax.ShapeDtypeStruct((N, 1), _f32),
  )
  return pl.pallas_call(_dense_body, out_shape=out_shape)(
      xa, xb, ws_ab, aas_ab.reshape(H, 1), wd_ab, ad_ab.reshape(H, 1),
      ws_ba, aas_ba.reshape(H, 1), wd_ba, ad_ba.reshape(H, 1))


def _combine(u):
  return u[:, :H] / (u[:, H:H + 1] + 1e-16)


def _combine_dense_body(u_pv_ref, u_vp_ref, b_pv_ref, b_vp_ref,
                        ws_ab_ref, as_ab_ref, wd_ab_ref, ad_ab_ref,
                        ws_ba_ref, as_ba_ref, wd_ba_ref, ad_ba_ref,
                        hs_ab_ref, ss_ab_ref, sd_ab_ref,
                        hs_ba_ref, ss_ba_ref, sd_ba_ref):
  h_v = jnp.maximum(_combine(u_pv_ref[...]) + b_pv_ref[...], 0.0)
  h_p = jnp.maximum(_combine(u_vp_ref[...]) + b_vp_ref[...], 0.0)
  hs_ab = _dot(h_p, ws_ab_ref[...])
  hs_ab_ref[...] = hs_ab
  ss_ab_ref[...] = _dot(hs_ab, as_ab_ref[...])
  sd_ab_ref[...] = _dot(_dot(h_v, wd_ab_ref[...]), ad_ab_ref[...])
  hs_ba = _dot(h_v, ws_ba_ref[...])
  hs_ba_ref[...] = hs_ba
  ss_ba_ref[...] = _dot(hs_ba, as_ba_ref[...])
  sd_ba_ref[...] = _dot(_dot(h_p, wd_ba_ref[...]), ad_ba_ref[...])


def _combine_dense(u_pv, u_vp, b_pv, b_vp,
                   ws_ab, aas_ab, wd_ab, ad_ab, ws_ba, aas_ba, wd_ba, ad_ba):
  out_shape = (
      jax.ShapeDtypeStruct((N, H), _f32),
      jax.ShapeDtypeStruct((N, 1), _f32),
      jax.ShapeDtypeStruct((N, 1), _f32),
      jax.ShapeDtypeStruct((N, H), _f32),
      jax.ShapeDtypeStruct((N, 1), _f32),
      jax.ShapeDtypeStruct((N, 1), _f32),
  )
  return pl.pallas_call(_combine_dense_body, out_shape=out_shape)(
      u_pv, u_vp, b_pv.reshape(1, H), b_vp.reshape(1, H),
      ws_ab, aas_ab.reshape(H, 1), wd_ab, ad_ab.reshape(H, 1),
      ws_ba, aas_ba.reshape(H, 1), wd_ba, ad_ba.reshape(H, 1))


def _final_body(u_pv_ref, u_vp_ref, b_pv_ref, b_vp_ref,
                hw_p_ref, hb_p_ref, hw_v_ref, hb_v_ref,
                out_p_ref, out_v_ref):
  h_v2 = jnp.maximum(_combine(u_pv_ref[...]) + b_pv_ref[...], 0.0)
  h_p2 = jnp.maximum(_combine(u_vp_ref[...]) + b_vp_ref[...], 0.0)
  out_p_ref[...] = _dot(h_p2, hw_p_ref[...]) + hb_p_ref[...]
  out_v_ref[...] = _dot(h_v2, hw_v_ref[...]) + hb_v_ref[...]


def _final(u_pv, u_vp, b_pv, b_vp, head_p_W, head_p_b, head_v_W, head_v_b):
  out_shape = (
      jax.ShapeDtypeStruct((N, OUT), _f32),
      jax.ShapeDtypeStruct((N, H), _f32),
  )
  return pl.pallas_call(_final_body, out_shape=out_shape)(
      u_pv, u_vp, b_pv.reshape(1, H), b_vp.reshape(1, H),
      head_p_W, head_p_b.reshape(1, OUT), head_v_W, head_v_b.reshape(1, H))


def _edge_body(src_pv, dst_pv, src_vp, dst_vp,
               ss_pv, sd_pv, ss_vp, sd_vp, hs_pv, hs_vp,
               u_pv, u_vp,
               ss_t, sd_t, idx_s, idx_d, ex_c, rows_c, stag_c, u_sp, sem):
  cid = lax.axis_index("c")
  sid = lax.axis_index("s")

  def run_rel(src2d, dst2d, ss_hbm, sd_hbm, hs_hbm, u_hbm):
    pltpu.sync_copy(ss_hbm, ss_t)
    pltpu.sync_copy(sd_hbm, sd_t)

    zero16 = jnp.zeros((16,), _f32)

    @pl.loop(0, CHUNK)
    def _(r):
      for q in range(ROW // 16):
        stag_c[r, pl.ds(q * 16, 16)] = zero16

    base_r = sid * RPW
    pltpu.sync_copy(stag_c, u_sp.at[pl.ds(base_r, CHUNK)])
    pltpu.sync_copy(stag_c.at[pl.ds(0, RPW - CHUNK)],
                    u_sp.at[pl.ds(base_r + CHUNK, RPW - CHUNK)])
    plsc.subcore_barrier()

    unit16 = (lax.iota(jnp.int32, 16) == 0).astype(_f32)

    @pl.loop(0, NCHUNK)
    def _(c):
      row0 = sid * (EPW // SUB) + c * NSUB
      pltpu.sync_copy(src2d.at[pl.ds(row0, NSUB)], idx_s)
      pltpu.sync_copy(dst2d.at[pl.ds(row0, NSUB)], idx_d)

      descs = [
          pltpu.async_copy(hs_hbm.at[idx_s.at[j]],
                           rows_c.at[pl.ds(j * SUB, SUB)], sem)
          for j in range(NSUB)
      ]
      for de in descs:
        de.wait()

      for j in range(NSUB):
        @pl.loop(0, SUB, step=16)
        def _(k, j=j):
          s16 = idx_s[j, pl.ds(k, 16)]
          d16 = idx_d[j, pl.ds(k, 16)]
          e = plsc.load_gather(ss_t, [s16]) + plsc.load_gather(sd_t, [d16])
          e = jnp.maximum(e, e * 0.2)
          ex_c[pl.ds(j * SUB + k, 16)] = jnp.exp(e)

      @pl.loop(0, CHUNK, step=16)
      def _(g):
        exv = ex_c[pl.ds(g, 16)]
        for t in range(16):
          exr = exv[t]
          r = g + t
          for q in range(H // 16):
            stag_c[r, pl.ds(q * 16, 16)] = rows_c[r, pl.ds(q * 16, 16)] * exr
          stag_c[r, pl.ds(H, 16)] = unit16 * exr

      for j in range(NSUB):
        pltpu.sync_copy(stag_c.at[pl.ds(j * SUB, SUB)],
                        u_sp.at[idx_d.at[j]], add=True)

    plsc.subcore_barrier()
    pltpu.sync_copy(u_sp.at[pl.ds(base_r, RPW)], u_hbm.at[pl.ds(base_r, RPW)])

  @pl.when(cid == 0)
  def _():
    run_rel(src_pv, dst_pv, ss_pv, sd_pv, hs_pv, u_pv)

  @pl.when(cid == 1)
  def _():
    run_rel(src_vp, dst_vp, ss_vp, sd_vp, hs_vp, u_vp)


@functools.partial(jax.jit, static_argnames=())
def _edge_pass(src_pv, dst_pv, src_vp, dst_vp,
               ss_pv, sd_pv, ss_vp, sd_vp, hs_pv, hs_vp):
  mesh = plsc.VectorSubcoreMesh(core_axis_name="c", subcore_axis_name="s")
  kern = pl.kernel(
      _edge_body,
      out_type=(jax.ShapeDtypeStruct((N, ROW), _f32),
                jax.ShapeDtypeStruct((N, ROW), _f32)),
      mesh=mesh,
      compiler_params=pltpu.CompilerParams(use_tc_tiling_on_sc=False,
                                           needs_layout_passes=False),
      scratch_types=[
          pltpu.VMEM((N,), _f32),
          pltpu.VMEM((N,), _f32),
          pltpu.VMEM((NSUB, SUB), jnp.int32),
          pltpu.VMEM((NSUB, SUB), jnp.int32),
          pltpu.VMEM((CHUNK,), _f32),
          pltpu.VMEM((CHUNK, H), _f32),
          pltpu.VMEM((CHUNK, ROW), _f32),
          pltpu.VMEM_SHARED((N, ROW), _f32),
          pltpu.SemaphoreType.DMA,
      ],
  )
  return kern(src_pv, dst_pv, src_vp, dst_vp,
              ss_pv, sd_pv, ss_vp, sd_vp, hs_pv, hs_vp)


def kernel(x_patient, x_visit, edge_index_pv, edge_index_vp,
           l1_pv_Wsrc, l1_pv_Wdst, l1_pv_asrc, l1_pv_adst, l1_pv_b,
           l1_vp_Wsrc, l1_vp_Wdst, l1_vp_asrc, l1_vp_adst, l1_vp_b,
           l2_pv_Wsrc, l2_pv_Wdst, l2_pv_asrc, l2_pv_adst, l2_pv_b,
           l2_vp_Wsrc, l2_vp_Wdst, l2_vp_asrc, l2_vp_adst, l2_vp_b,
           head_p_W, head_p_b, head_v_W, head_v_b):
  src_pv = edge_index_pv[0].reshape(E // SUB, SUB)
  dst_pv = edge_index_pv[1].reshape(E // SUB, SUB)
  src_vp = edge_index_vp[0].reshape(E // SUB, SUB)
  dst_vp = edge_index_vp[1].reshape(E // SUB, SUB)

  hs1_pv, ss1_pv, sd1_pv, hs1_vp, ss1_vp, sd1_vp = _dense_pair(
      x_patient, x_visit,
      l1_pv_Wsrc, l1_pv_asrc, l1_pv_Wdst, l1_pv_adst,
      l1_vp_Wsrc, l1_vp_asrc, l1_vp_Wdst, l1_vp_adst)

  u1_pv, u1_vp = _edge_pass(
      src_pv, dst_pv, src_vp, dst_vp,
      ss1_pv.reshape(N), sd1_pv.reshape(N),
      ss1_vp.reshape(N), sd1_vp.reshape(N), hs1_pv, hs1_vp)

  hs2_pv, ss2_pv, sd2_pv, hs2_vp, ss2_vp, sd2_vp = _combine_dense(
      u1_pv, u1_vp, l1_pv_b, l1_vp_b,
      l2_pv_Wsrc, l2_pv_asrc, l2_pv_Wdst, l2_pv_adst,
      l2_vp_Wsrc, l2_vp_asrc, l2_vp_Wdst, l2_vp_adst)

  u2_pv, u2_vp = _edge_pass(
      src_pv, dst_pv, src_vp, dst_vp,
      ss2_pv.reshape(N), sd2_pv.reshape(N),
      ss2_vp.reshape(N), sd2_vp.reshape(N), hs2_pv, hs2_vp)

  out_p, out_v = _final(u2_pv, u2_vp, l2_pv_b, l2_vp_b,
                        head_p_W, head_p_b, head_v_W, head_v_b)
  return (out_p, out_v)


# speedup vs baseline: 33.1627x; 1.3298x over previous
import functools

import jax
import jax.numpy as jnp
from jax import lax
from jax.experimental import pallas as pl
from jax.experimental.pallas import tpu as pltpu
from jax.experimental.pallas import tpu_sc as plsc

N = 10000
E = 160000
D = 128
H = 64
OUT = 32
ROW = 80
SUB = 80
NSUB = 1
CHUNK = SUB * NSUB
NW = 16
EPW = E // NW
NCHUNK = EPW // CHUNK
IDXR = EPW // SUB
RPW = N // NW

_f32 = jnp.float32


def _dot(a, b):
  return jnp.dot(a, b, preferred_element_type=_f32)


def _dense_pair_core(xa, xb, ws_ab, as_ab, wd_ab, ad_ab,
                     ws_ba, as_ba, wd_ba, ad_ba,
                     hs_ab_ref, ss_ab_ref, sd_ab_ref,
                     hs_ba_ref, ss_ba_ref, sd_ba_ref):
  hs_ab = _dot(xa, ws_ab)
  hs_ab_ref[...] = hs_ab
  ss_ab_ref[...] = _dot(hs_ab, as_ab)
  sd_ab_ref[...] = _dot(_dot(xb, wd_ab), ad_ab)
  hs_ba = _dot(xb, ws_ba)
  hs_ba_ref[...] = hs_ba
  ss_ba_ref[...] = _dot(hs_ba, as_ba)
  sd_ba_ref[...] = _dot(_dot(xa, wd_ba), ad_ba)


def _dense_body(xa_ref, xb_ref,
                ws_ab_ref, as_ab_ref, wd_ab_ref, ad_ab_ref,
                ws_ba_ref, as_ba_ref, wd_ba_ref, ad_ba_ref,
                *out_refs):
  _dense_pair_core(xa_ref[...], xb_ref[...],
                   ws_ab_ref[...], as_ab_ref[...], wd_ab_ref[...],
                   ad_ab_ref[...], ws_ba_ref[...], as_ba_ref[...],
                   wd_ba_ref[...], ad_ba_ref[...], *out_refs)


_PAIR_OUT = (
    jax.ShapeDtypeStruct((N, H), _f32),
    jax.ShapeDtypeStruct((N, 1), _f32),
    jax.ShapeDtypeStruct((N, 1), _f32),
    jax.ShapeDtypeStruct((N, H), _f32),
    jax.ShapeDtypeStruct((N, 1), _f32),
    jax.ShapeDtypeStruct((N, 1), _f32),
)


def _dense_pair(xa, xb, ws_ab, aas_ab, wd_ab, ad_ab, ws_ba, aas_ba, wd_ba, ad_ba):
  return pl.pallas_call(_dense_body, out_shape=_PAIR_OUT)(
      xa, xb, ws_ab, aas_ab.reshape(H, 1), wd_ab, ad_ab.reshape(H, 1),
      ws_ba, aas_ba.reshape(H, 1), wd_ba, ad_ba.reshape(H, 1))


def _combine(u):
  return u[:, :H] / (u[:, H:H + 1] + 1e-16)


def _combine_dense_body(u_pv_ref, u_vp_ref, b_pv_ref, b_vp_ref,
                        ws_ab_ref, as_ab_ref, wd_ab_ref, ad_ab_ref,
                        ws_ba_ref, as_ba_ref, wd_ba_ref, ad_ba_ref,
                        *out_refs):
  h_v = jnp.maximum(_combine(u_pv_ref[...]) + b_pv_ref[...], 0.0)
  h_p = jnp.maximum(_combine(u_vp_ref[...]) + b_vp_ref[...], 0.0)
  _dense_pair_core(h_p, h_v,
                   ws_ab_ref[...], as_ab_ref[...], wd_ab_ref[...],
                   ad_ab_ref[...], ws_ba_ref[...], as_ba_ref[...],
                   wd_ba_ref[...], ad_ba_ref[...], *out_refs)


def _combine_dense(u_pv, u_vp, b_pv, b_vp,
                   ws_ab, aas_ab, wd_ab, ad_ab, ws_ba, aas_ba, wd_ba, ad_ba):
  return pl.pallas_call(_combine_dense_body, out_shape=_PAIR_OUT)(
      u_pv, u_vp, b_pv.reshape(1, H), b_vp.reshape(1, H),
      ws_ab, aas_ab.reshape(H, 1), wd_ab, ad_ab.reshape(H, 1),
      ws_ba, aas_ba.reshape(H, 1), wd_ba, ad_ba.reshape(H, 1))


def _final_body(u_pv_ref, u_vp_ref, b_pv_ref, b_vp_ref,
                hw_p_ref, hb_p_ref, hw_v_ref, hb_v_ref,
                out_p_ref, out_v_ref):
  h_v2 = jnp.maximum(_combine(u_pv_ref[...]) + b_pv_ref[...], 0.0)
  h_p2 = jnp.maximum(_combine(u_vp_ref[...]) + b_vp_ref[...], 0.0)
  out_p_ref[...] = _dot(h_p2, hw_p_ref[...]) + hb_p_ref[...]
  out_v_ref[...] = _dot(h_v2, hw_v_ref[...]) + hb_v_ref[...]


def _final(u_pv, u_vp, b_pv, b_vp, head_p_W, head_p_b, head_v_W, head_v_b):
  out_shape = (
      jax.ShapeDtypeStruct((N, OUT), _f32),
      jax.ShapeDtypeStruct((N, H), _f32),
  )
  return pl.pallas_call(_final_body, out_shape=out_shape)(
      u_pv, u_vp, b_pv.reshape(1, H), b_vp.reshape(1, H),
      head_p_W, head_p_b.reshape(1, OUT), head_v_W, head_v_b.reshape(1, H))


def _edge_body(src_pv, dst_pv, src_vp, dst_vp,
               ss_pv, sd_pv, ss_vp, sd_vp, hs_pv, hs_vp,
               u_pv, u_vp,
               ss_t, sd_t, src_w, dst_w, rows0, rows1, stag0, stag1, u_sp,
               sg0, sg1, sc0, sc1):
  cid = lax.axis_index("c")
  sid = lax.axis_index("s")

  def run_rel(src2d, dst2d, ss_hbm, sd_hbm, hsx_hbm, u_hbm):
    pltpu.sync_copy(ss_hbm, ss_t)
    pltpu.sync_copy(sd_hbm, sd_t)
    pltpu.sync_copy(src2d.at[pl.ds(sid * IDXR, IDXR)], src_w)
    pltpu.sync_copy(dst2d.at[pl.ds(sid * IDXR, IDXR)], dst_w)

    zero16 = jnp.zeros((16,), _f32)

    @pl.loop(0, CHUNK)
    def _(r):
      for q in range(ROW // 16):
        stag0[r, pl.ds(q * 16, 16)] = zero16

    base_r = sid * RPW
    for z in range(RPW // CHUNK):
      pltpu.sync_copy(stag0, u_sp.at[pl.ds(base_r + z * CHUNK, CHUNK)])
    if RPW % CHUNK:
      pltpu.sync_copy(stag0.at[pl.ds(0, RPW % CHUNK)],
                      u_sp.at[pl.ds(base_r + (RPW // CHUNK) * CHUNK,
                                    RPW % CHUNK)])
    plsc.subcore_barrier()

    unit16 = (lax.iota(jnp.int32, 16) == 0).astype(_f32)

    def g_descs(c, buf, sem):
      return [pltpu.make_async_copy(hsx_hbm.at[src_w.at[c * NSUB + j]],
                                    buf.at[pl.ds(j * SUB, SUB)], sem)
              for j in range(NSUB)]

    def s_descs(c, buf, sem):
      return [pltpu.make_async_copy(buf.at[pl.ds(j * SUB, SUB)],
                                    u_sp.at[dst_w.at[c * NSUB + j]], sem)
              for j in range(NSUB)]

    def fire_gather(c, buf, sem):
      for j in range(NSUB):
        pltpu.async_copy(hsx_hbm.at[src_w.at[c * NSUB + j]],
                         buf.at[pl.ds(j * SUB, SUB)], sem)

    def wait_gather(c, buf, sem):
      for de in g_descs(c, buf, sem):
        de.wait()

    def fire_scatter(c, buf, sem):
      for j in range(NSUB):
        pltpu.async_copy(buf.at[pl.ds(j * SUB, SUB)],
                         u_sp.at[dst_w.at[c * NSUB + j]], sem, add=True)

    def wait_scatter(c, buf, sem):
      for de in s_descs(c, buf, sem):
        de.wait()

    def compute(c, rows, stag):
      for j in range(NSUB):
        @pl.loop(0, SUB, step=16)
        def _(k, j=j):
          ri = c * NSUB + j
          s16 = src_w[ri, pl.ds(k, 16)]
          d16 = dst_w[ri, pl.ds(k, 16)]
          e = plsc.load_gather(ss_t, [s16]) + plsc.load_gather(sd_t, [d16])
          e = jnp.maximum(e, e * 0.2)
          ex = jnp.exp(e)
          for t in range(16):
            r = j * SUB + k + t
            exr = ex[t]
            for q in range(H // 16):
              stag[r, pl.ds(q * 16, 16)] = rows[r, pl.ds(q * 16, 16)] * exr
            stag[r, pl.ds(H, 16)] = unit16 * exr

    fire_gather(0, rows0, sg0)

    @pl.loop(0, NCHUNK - 1, step=2)
    def _(cc):
      fire_gather(cc + 1, rows1, sg1)
      wait_gather(cc, rows0, sg0)
      @pl.when(cc > 0)
      def _():
        wait_scatter(cc - 2, stag0, sc0)
      compute(cc, rows0, stag0)
      fire_scatter(cc, stag0, sc0)
      fire_gather(cc + 2, rows0, sg0)
      wait_gather(cc + 1, rows1, sg1)
      @pl.when(cc > 0)
      def _():
        wait_scatter(cc - 1, stag1, sc1)
      compute(cc + 1, rows1, stag1)
      fire_scatter(cc + 1, stag1, sc1)

    wait_gather(NCHUNK - 1, rows0, sg0)
    wait_scatter(NCHUNK - 3, stag0, sc0)
    compute(NCHUNK - 1, rows0, stag0)
    fire_scatter(NCHUNK - 1, stag0, sc0)
    wait_scatter(NCHUNK - 2, stag1, sc1)
    wait_scatter(NCHUNK - 1, stag0, sc0)

    plsc.subcore_barrier()
    pltpu.sync_copy(u_sp.at[pl.ds(base_r, RPW)], u_hbm.at[pl.ds(base_r, RPW)])

  @pl.when(cid == 0)
  def _():
    run_rel(src_pv, dst_pv, ss_pv, sd_pv, hs_pv, u_pv)

  @pl.when(cid == 1)
  def _():
    run_rel(src_vp, dst_vp, ss_vp, sd_vp, hs_vp, u_vp)


def _edge_pass(src_pv, dst_pv, src_vp, dst_vp,
               ss_pv, sd_pv, ss_vp, sd_vp, hs_pv, hs_vp):
  mesh = plsc.VectorSubcoreMesh(core_axis_name="c", subcore_axis_name="s")
  kern = pl.kernel(
      _edge_body,
      out_type=(jax.ShapeDtypeStruct((N, ROW), _f32),
                jax.ShapeDtypeStruct((N, ROW), _f32)),
      mesh=mesh,
      compiler_params=pltpu.CompilerParams(use_tc_tiling_on_sc=False,
                                           needs_layout_passes=False),
      scratch_types=[
          pltpu.VMEM((N,), _f32),
          pltpu.VMEM((N,), _f32),
          pltpu.VMEM((IDXR, SUB), jnp.int32),
          pltpu.VMEM((IDXR, SUB), jnp.int32),
          pltpu.VMEM((CHUNK, H), _f32),
          pltpu.VMEM((CHUNK, H), _f32),
          pltpu.VMEM((CHUNK, ROW), _f32),
          pltpu.VMEM((CHUNK, ROW), _f32),
          pltpu.VMEM_SHARED((N, ROW), _f32),
          pltpu.SemaphoreType.DMA,
          pltpu.SemaphoreType.DMA,
          pltpu.SemaphoreType.DMA,
          pltpu.SemaphoreType.DMA,
      ],
  )
  return kern(src_pv, dst_pv, src_vp, dst_vp,
              ss_pv, sd_pv, ss_vp, sd_vp, hs_pv, hs_vp)


def kernel(x_patient, x_visit, edge_index_pv, edge_index_vp,
           l1_pv_Wsrc, l1_pv_Wdst, l1_pv_asrc, l1_pv_adst, l1_pv_b,
           l1_vp_Wsrc, l1_vp_Wdst, l1_vp_asrc, l1_vp_adst, l1_vp_b,
           l2_pv_Wsrc, l2_pv_Wdst, l2_pv_asrc, l2_pv_adst, l2_pv_b,
           l2_vp_Wsrc, l2_vp_Wdst, l2_vp_asrc, l2_vp_adst, l2_vp_b,
           head_p_W, head_p_b, head_v_W, head_v_b):
  src_pv = edge_index_pv[0].reshape(E // SUB, SUB)
  dst_pv = edge_index_pv[1].reshape(E // SUB, SUB)
  src_vp = edge_index_vp[0].reshape(E // SUB, SUB)
  dst_vp = edge_index_vp[1].reshape(E // SUB, SUB)

  hs1_pv, ss1_pv, sd1_pv, hs1_vp, ss1_vp, sd1_vp = _dense_pair(
      x_patient, x_visit,
      l1_pv_Wsrc, l1_pv_asrc, l1_pv_Wdst, l1_pv_adst,
      l1_vp_Wsrc, l1_vp_asrc, l1_vp_Wdst, l1_vp_adst)

  u1_pv, u1_vp = _edge_pass(
      src_pv, dst_pv, src_vp, dst_vp,
      ss1_pv.reshape(N), sd1_pv.reshape(N),
      ss1_vp.reshape(N), sd1_vp.reshape(N), hs1_pv, hs1_vp)

  hs2_pv, ss2_pv, sd2_pv, hs2_vp, ss2_vp, sd2_vp = _combine_dense(
      u1_pv, u1_vp, l1_pv_b, l1_vp_b,
      l2_pv_Wsrc, l2_pv_asrc, l2_pv_Wdst, l2_pv_adst,
      l2_vp_Wsrc, l2_vp_asrc, l2_vp_Wdst, l2_vp_adst)

  u2_pv, u2_vp = _edge_pass(
      src_pv, dst_pv, src_vp, dst_vp,
      ss2_pv.reshape(N), sd2_pv.reshape(N),
      ss2_vp.reshape(N), sd2_vp.reshape(N), hs2_pv, hs2_vp)

  out_p, out_v = _final(u2_pv, u2_vp, l2_pv_b, l2_vp_b,
                        head_p_W, head_p_b, head_v_W, head_v_b)
  return (out_p, out_v)


# speedup vs baseline: 35.0804x; 1.0578x over previous
import functools

import jax
import jax.numpy as jnp
from jax import lax
from jax.experimental import pallas as pl
from jax.experimental.pallas import tpu as pltpu
from jax.experimental.pallas import tpu_sc as plsc

N = 10000
E = 160000
D = 128
H = 64
OUT = 32
ROW = 80
SUB = 80
NSUB = 1
CHUNK = SUB * NSUB
NW = 16
EPW = E // NW
NCHUNK = EPW // CHUNK
IDXR = EPW // SUB
RPW = N // NW

_f32 = jnp.float32


def _dot(a, b):
  return jnp.dot(a, b, preferred_element_type=_f32)


def _dense_pair_core(xa, xb, ws_ab, as_ab, wd_ab, ad_ab,
                     ws_ba, as_ba, wd_ba, ad_ba,
                     hs_ab_ref, ss_ab_ref, sd_ab_ref,
                     hs_ba_ref, ss_ba_ref, sd_ba_ref):
  hs_ab = _dot(xa, ws_ab)
  hs_ab_ref[...] = hs_ab
  ss_ab_ref[...] = jnp.sum(hs_ab * as_ab, axis=1)
  sd_ab_ref[...] = jnp.sum(_dot(xb, wd_ab) * ad_ab, axis=1)
  hs_ba = _dot(xb, ws_ba)
  hs_ba_ref[...] = hs_ba
  ss_ba_ref[...] = jnp.sum(hs_ba * as_ba, axis=1)
  sd_ba_ref[...] = jnp.sum(_dot(xa, wd_ba) * ad_ba, axis=1)


def _dense_body(xa_ref, xb_ref,
                ws_ab_ref, as_ab_ref, wd_ab_ref, ad_ab_ref,
                ws_ba_ref, as_ba_ref, wd_ba_ref, ad_ba_ref,
                *out_refs):
  _dense_pair_core(xa_ref[...], xb_ref[...],
                   ws_ab_ref[...], as_ab_ref[...], wd_ab_ref[...],
                   ad_ab_ref[...], ws_ba_ref[...], as_ba_ref[...],
                   wd_ba_ref[...], ad_ba_ref[...], *out_refs)


_PAIR_OUT = (
    jax.ShapeDtypeStruct((N, H), _f32),
    jax.ShapeDtypeStruct((N,), _f32),
    jax.ShapeDtypeStruct((N,), _f32),
    jax.ShapeDtypeStruct((N, H), _f32),
    jax.ShapeDtypeStruct((N,), _f32),
    jax.ShapeDtypeStruct((N,), _f32),
)


def _dense_pair(xa, xb, ws_ab, aas_ab, wd_ab, ad_ab, ws_ba, aas_ba, wd_ba, ad_ba):
  return pl.pallas_call(_dense_body, out_shape=_PAIR_OUT)(
      xa, xb, ws_ab, aas_ab.reshape(1, H), wd_ab, ad_ab.reshape(1, H),
      ws_ba, aas_ba.reshape(1, H), wd_ba, ad_ba.reshape(1, H))


def _combine(u):
  return u[:, :H] / (u[:, H:H + 1] + 1e-16)


def _combine_dense_body(u_pv_ref, u_vp_ref, b_pv_ref, b_vp_ref,
                        ws_ab_ref, as_ab_ref, wd_ab_ref, ad_ab_ref,
                        ws_ba_ref, as_ba_ref, wd_ba_ref, ad_ba_ref,
                        *out_refs):
  h_v = jnp.maximum(_combine(u_pv_ref[...]) + b_pv_ref[...], 0.0)
  h_p = jnp.maximum(_combine(u_vp_ref[...]) + b_vp_ref[...], 0.0)
  _dense_pair_core(h_p, h_v,
                   ws_ab_ref[...], as_ab_ref[...], wd_ab_ref[...],
                   ad_ab_ref[...], ws_ba_ref[...], as_ba_ref[...],
                   wd_ba_ref[...], ad_ba_ref[...], *out_refs)


def _combine_dense(u_pv, u_vp, b_pv, b_vp,
                   ws_ab, aas_ab, wd_ab, ad_ab, ws_ba, aas_ba, wd_ba, ad_ba):
  return pl.pallas_call(_combine_dense_body, out_shape=_PAIR_OUT)(
      u_pv, u_vp, b_pv.reshape(1, H), b_vp.reshape(1, H),
      ws_ab, aas_ab.reshape(1, H), wd_ab, ad_ab.reshape(1, H),
      ws_ba, aas_ba.reshape(1, H), wd_ba, ad_ba.reshape(1, H))


def _final_body(u_pv_ref, u_vp_ref, b_pv_ref, b_vp_ref,
                hw_p_ref, hb_p_ref, hw_v_ref, hb_v_ref,
                out_p_ref, out_v_ref):
  h_v2 = jnp.maximum(_combine(u_pv_ref[...]) + b_pv_ref[...], 0.0)
  h_p2 = jnp.maximum(_combine(u_vp_ref[...]) + b_vp_ref[...], 0.0)
  out_p_ref[...] = _dot(h_p2, hw_p_ref[...]) + hb_p_ref[...]
  out_v_ref[...] = _dot(h_v2, hw_v_ref[...]) + hb_v_ref[...]


def _final(u_pv, u_vp, b_pv, b_vp, head_p_W, head_p_b, head_v_W, head_v_b):
  out_shape = (
      jax.ShapeDtypeStruct((N, OUT), _f32),
      jax.ShapeDtypeStruct((N, H), _f32),
  )
  return pl.pallas_call(_final_body, out_shape=out_shape)(
      u_pv, u_vp, b_pv.reshape(1, H), b_vp.reshape(1, H),
      head_p_W, head_p_b.reshape(1, OUT), head_v_W, head_v_b.reshape(1, H))


def _edge_body(src_pv, dst_pv, src_vp, dst_vp,
               ss_pv, sd_pv, ss_vp, sd_vp, hs_pv, hs_vp,
               u_pv, u_vp,
               ss_t, sd_t, src_w, dst_w, rows0, rows1, stag0, stag1, u_sp,
               sg0, sg1, sc0, sc1):
  cid = lax.axis_index("c")
  sid = lax.axis_index("s")

  def run_rel(src2d, dst2d, ss_hbm, sd_hbm, hsx_hbm, u_hbm):
    pltpu.sync_copy(ss_hbm, ss_t)
    pltpu.sync_copy(sd_hbm, sd_t)
    pltpu.sync_copy(src2d.at[pl.ds(sid * IDXR, IDXR)], src_w)
    pltpu.sync_copy(dst2d.at[pl.ds(sid * IDXR, IDXR)], dst_w)

    zero16 = jnp.zeros((16,), _f32)

    @pl.loop(0, CHUNK)
    def _(r):
      for q in range(ROW // 16):
        stag0[r, pl.ds(q * 16, 16)] = zero16

    base_r = sid * RPW
    for z in range(RPW // CHUNK):
      pltpu.sync_copy(stag0, u_sp.at[pl.ds(base_r + z * CHUNK, CHUNK)])
    if RPW % CHUNK:
      pltpu.sync_copy(stag0.at[pl.ds(0, RPW % CHUNK)],
                      u_sp.at[pl.ds(base_r + (RPW // CHUNK) * CHUNK,
                                    RPW % CHUNK)])
    plsc.subcore_barrier()

    unit16 = (lax.iota(jnp.int32, 16) == 0).astype(_f32)

    def g_descs(c, buf, sem):
      return [pltpu.make_async_copy(hsx_hbm.at[src_w.at[c * NSUB + j]],
                                    buf.at[pl.ds(j * SUB, SUB)], sem)
              for j in range(NSUB)]

    def s_descs(c, buf, sem):
      return [pltpu.make_async_copy(buf.at[pl.ds(j * SUB, SUB)],
                                    u_sp.at[dst_w.at[c * NSUB + j]], sem)
              for j in range(NSUB)]

    def fire_gather(c, buf, sem):
      for j in range(NSUB):
        pltpu.async_copy(hsx_hbm.at[src_w.at[c * NSUB + j]],
                         buf.at[pl.ds(j * SUB, SUB)], sem)

    def wait_gather(c, buf, sem):
      for de in g_descs(c, buf, sem):
        de.wait()

    def fire_scatter(c, buf, sem):
      for j in range(NSUB):
        pltpu.async_copy(buf.at[pl.ds(j * SUB, SUB)],
                         u_sp.at[dst_w.at[c * NSUB + j]], sem, add=True)

    def wait_scatter(c, buf, sem):
      for de in s_descs(c, buf, sem):
        de.wait()

    def compute(c, rows, stag):
      for j in range(NSUB):
        @pl.loop(0, SUB, step=16)
        def _(k, j=j):
          ri = c * NSUB + j
          s16 = src_w[ri, pl.ds(k, 16)]
          d16 = dst_w[ri, pl.ds(k, 16)]
          e = plsc.load_gather(ss_t, [s16]) + plsc.load_gather(sd_t, [d16])
          e = jnp.maximum(e, e * 0.2)
          ex = jnp.exp(e)
          for t in range(16):
            r = j * SUB + k + t
            exr = ex[t]
            for q in range(H // 16):
              stag[r, pl.ds(q * 16, 16)] = rows[r, pl.ds(q * 16, 16)] * exr
            stag[r, pl.ds(H, 16)] = unit16 * exr

    fire_gather(0, rows0, sg0)

    @pl.loop(0, NCHUNK - 1, step=2)
    def _(cc):
      fire_gather(cc + 1, rows1, sg1)
      wait_gather(cc, rows0, sg0)
      @pl.when(cc > 0)
      def _():
        wait_scatter(cc - 2, stag0, sc0)
      compute(cc, rows0, stag0)
      fire_scatter(cc, stag0, sc0)
      fire_gather(cc + 2, rows0, sg0)
      wait_gather(cc + 1, rows1, sg1)
      @pl.when(cc > 0)
      def _():
        wait_scatter(cc - 1, stag1, sc1)
      compute(cc + 1, rows1, stag1)
      fire_scatter(cc + 1, stag1, sc1)

    wait_gather(NCHUNK - 1, rows0, sg0)
    wait_scatter(NCHUNK - 3, stag0, sc0)
    compute(NCHUNK - 1, rows0, stag0)
    fire_scatter(NCHUNK - 1, stag0, sc0)
    wait_scatter(NCHUNK - 2, stag1, sc1)
    wait_scatter(NCHUNK - 1, stag0, sc0)

    plsc.subcore_barrier()
    pltpu.sync_copy(u_sp.at[pl.ds(base_r, RPW)], u_hbm.at[pl.ds(base_r, RPW)])

  @pl.when(cid == 0)
  def _():
    run_rel(src_pv, dst_pv, ss_pv, sd_pv, hs_pv, u_pv)

  @pl.when(cid == 1)
  def _():
    run_rel(src_vp, dst_vp, ss_vp, sd_vp, hs_vp, u_vp)


def _edge_pass(src_pv, dst_pv, src_vp, dst_vp,
               ss_pv, sd_pv, ss_vp, sd_vp, hs_pv, hs_vp):
  mesh = plsc.VectorSubcoreMesh(core_axis_name="c", subcore_axis_name="s")
  kern = pl.kernel(
      _edge_body,
      out_type=(jax.ShapeDtypeStruct((N, ROW), _f32),
                jax.ShapeDtypeStruct((N, ROW), _f32)),
      mesh=mesh,
      compiler_params=pltpu.CompilerParams(use_tc_tiling_on_sc=False,
                                           needs_layout_passes=False),
      scratch_types=[
          pltpu.VMEM((N,), _f32),
          pltpu.VMEM((N,), _f32),
          pltpu.VMEM((IDXR, SUB), jnp.int32),
          pltpu.VMEM((IDXR, SUB), jnp.int32),
          pltpu.VMEM((CHUNK, H), _f32),
          pltpu.VMEM((CHUNK, H), _f32),
          pltpu.VMEM((CHUNK, ROW), _f32),
          pltpu.VMEM((CHUNK, ROW), _f32),
          pltpu.VMEM_SHARED((N, ROW), _f32),
          pltpu.SemaphoreType.DMA,
          pltpu.SemaphoreType.DMA,
          pltpu.SemaphoreType.DMA,
          pltpu.SemaphoreType.DMA,
      ],
  )
  return kern(src_pv, dst_pv, src_vp, dst_vp,
              ss_pv, sd_pv, ss_vp, sd_vp, hs_pv, hs_vp)


def kernel(x_patient, x_visit, edge_index_pv, edge_index_vp,
           l1_pv_Wsrc, l1_pv_Wdst, l1_pv_asrc, l1_pv_adst, l1_pv_b,
           l1_vp_Wsrc, l1_vp_Wdst, l1_vp_asrc, l1_vp_adst, l1_vp_b,
           l2_pv_Wsrc, l2_pv_Wdst, l2_pv_asrc, l2_pv_adst, l2_pv_b,
           l2_vp_Wsrc, l2_vp_Wdst, l2_vp_asrc, l2_vp_adst, l2_vp_b,
           head_p_W, head_p_b, head_v_W, head_v_b):
  src_pv = edge_index_pv[0].reshape(E // SUB, SUB)
  dst_pv = edge_index_pv[1].reshape(E // SUB, SUB)
  src_vp = edge_index_vp[0].reshape(E // SUB, SUB)
  dst_vp = edge_index_vp[1].reshape(E // SUB, SUB)

  hs1_pv, ss1_pv, sd1_pv, hs1_vp, ss1_vp, sd1_vp = _dense_pair(
      x_patient, x_visit,
      l1_pv_Wsrc, l1_pv_asrc, l1_pv_Wdst, l1_pv_adst,
      l1_vp_Wsrc, l1_vp_asrc, l1_vp_Wdst, l1_vp_adst)

  u1_pv, u1_vp = _edge_pass(
      src_pv, dst_pv, src_vp, dst_vp,
      ss1_pv, sd1_pv, ss1_vp, sd1_vp, hs1_pv, hs1_vp)

  hs2_pv, ss2_pv, sd2_pv, hs2_vp, ss2_vp, sd2_vp = _combine_dense(
      u1_pv, u1_vp, l1_pv_b, l1_vp_b,
      l2_pv_Wsrc, l2_pv_asrc, l2_pv_Wdst, l2_pv_adst,
      l2_vp_Wsrc, l2_vp_asrc, l2_vp_Wdst, l2_vp_adst)

  u2_pv, u2_vp = _edge_pass(
      src_pv, dst_pv, src_vp, dst_vp,
      ss2_pv, sd2_pv, ss2_vp, sd2_vp, hs2_pv, hs2_vp)

  out_p, out_v = _final(u2_pv, u2_vp, l2_pv_b, l2_vp_b,
                        head_p_W, head_p_b, head_v_W, head_v_b)
  return (out_p, out_v)


# speedup vs baseline: 53.3378x; 1.5204x over previous
import functools

import jax
import jax.numpy as jnp
from jax import lax
from jax.experimental import pallas as pl
from jax.experimental.pallas import tpu as pltpu
from jax.experimental.pallas import tpu_sc as plsc

N = 10000
E = 160000
D = 128
H = 64
OUT = 32
ROW = 80
SUB = 80
NSUB = 1
CHUNK = SUB * NSUB
NW = 16
EPW = E // NW
NCHUNK = EPW // CHUNK
IDXR = EPW // SUB
RPW = N // NW

_f32 = jnp.float32


def _dot(a, b):
  return jnp.dot(a, b, preferred_element_type=_f32)


def _dense_pair_core(xa, xb, ws_ab, as_ab, wd_ab, ad_ab,
                     ws_ba, as_ba, wd_ba, ad_ba,
                     hs_ab_ref, ss_ab_ref, sd_ab_ref,
                     hs_ba_ref, ss_ba_ref, sd_ba_ref):
  hs_ab = _dot(xa, ws_ab)
  hs_ab_ref[...] = hs_ab
  ss_ab_ref[...] = jnp.sum(hs_ab * as_ab, axis=1)
  sd_ab_ref[...] = jnp.sum(_dot(xb, wd_ab) * ad_ab, axis=1)
  hs_ba = _dot(xb, ws_ba)
  hs_ba_ref[...] = hs_ba
  ss_ba_ref[...] = jnp.sum(hs_ba * as_ba, axis=1)
  sd_ba_ref[...] = jnp.sum(_dot(xa, wd_ba) * ad_ba, axis=1)


def _dense_body(xa_ref, xb_ref,
                ws_ab_ref, as_ab_ref, wd_ab_ref, ad_ab_ref,
                ws_ba_ref, as_ba_ref, wd_ba_ref, ad_ba_ref,
                *out_refs):
  _dense_pair_core(xa_ref[...], xb_ref[...],
                   ws_ab_ref[...], as_ab_ref[...], wd_ab_ref[...],
                   ad_ab_ref[...], ws_ba_ref[...], as_ba_ref[...],
                   wd_ba_ref[...], ad_ba_ref[...], *out_refs)


_PAIR_OUT = (
    jax.ShapeDtypeStruct((N, H), _f32),
    jax.ShapeDtypeStruct((N,), _f32),
    jax.ShapeDtypeStruct((N,), _f32),
    jax.ShapeDtypeStruct((N, H), _f32),
    jax.ShapeDtypeStruct((N,), _f32),
    jax.ShapeDtypeStruct((N,), _f32),
)


def _dense_pair(xa, xb, ws_ab, aas_ab, wd_ab, ad_ab, ws_ba, aas_ba, wd_ba, ad_ba):
  return pl.pallas_call(_dense_body, out_shape=_PAIR_OUT)(
      xa, xb, ws_ab, aas_ab.reshape(1, H), wd_ab, ad_ab.reshape(1, H),
      ws_ba, aas_ba.reshape(1, H), wd_ba, ad_ba.reshape(1, H))


def _combine(u):
  return u[:, :H] / (u[:, H:H + 1] + 1e-16)


def _combine_dense_body(u_pv_ref, u_vp_ref, b_pv_ref, b_vp_ref,
                        ws_ab_ref, as_ab_ref, wd_ab_ref, ad_ab_ref,
                        ws_ba_ref, as_ba_ref, wd_ba_ref, ad_ba_ref,
                        *out_refs):
  h_v = jnp.maximum(_combine(u_pv_ref[...]) + b_pv_ref[...], 0.0)
  h_p = jnp.maximum(_combine(u_vp_ref[...]) + b_vp_ref[...], 0.0)
  _dense_pair_core(h_p, h_v,
                   ws_ab_ref[...], as_ab_ref[...], wd_ab_ref[...],
                   ad_ab_ref[...], ws_ba_ref[...], as_ba_ref[...],
                   wd_ba_ref[...], ad_ba_ref[...], *out_refs)


def _combine_dense(u_pv, u_vp, b_pv, b_vp,
                   ws_ab, aas_ab, wd_ab, ad_ab, ws_ba, aas_ba, wd_ba, ad_ba):
  return pl.pallas_call(_combine_dense_body, out_shape=_PAIR_OUT)(
      u_pv, u_vp, b_pv.reshape(1, H), b_vp.reshape(1, H),
      ws_ab, aas_ab.reshape(1, H), wd_ab, ad_ab.reshape(1, H),
      ws_ba, aas_ba.reshape(1, H), wd_ba, ad_ba.reshape(1, H))


def _final_body(u_pv_ref, u_vp_ref, b_pv_ref, b_vp_ref,
                hw_p_ref, hb_p_ref, hw_v_ref, hb_v_ref,
                out_p_ref, out_v_ref):
  h_v2 = jnp.maximum(_combine(u_pv_ref[...]) + b_pv_ref[...], 0.0)
  h_p2 = jnp.maximum(_combine(u_vp_ref[...]) + b_vp_ref[...], 0.0)
  out_p_ref[...] = _dot(h_p2, hw_p_ref[...]) + hb_p_ref[...]
  out_v_ref[...] = _dot(h_v2, hw_v_ref[...]) + hb_v_ref[...]


def _final(u_pv, u_vp, b_pv, b_vp, head_p_W, head_p_b, head_v_W, head_v_b):
  out_shape = (
      jax.ShapeDtypeStruct((N, OUT), _f32),
      jax.ShapeDtypeStruct((N, H), _f32),
  )
  return pl.pallas_call(_final_body, out_shape=out_shape)(
      u_pv, u_vp, b_pv.reshape(1, H), b_vp.reshape(1, H),
      head_p_W, head_p_b.reshape(1, OUT), head_v_W, head_v_b.reshape(1, H))


def _edge_body(src_pv, dst_pv, src_vp, dst_vp,
               ss_pv, sd_pv, ss_vp, sd_vp, hs_pv, hs_vp,
               u_pv, u_vp,
               ss_t, sd_t, src_w, dst_w, rows0, rows1, stag0, stag1, u_sp,
               sg0, sg1, sc0, sc1):
  cid = lax.axis_index("c")
  sid = lax.axis_index("s")

  def run_rel(src2d, dst2d, ss_hbm, sd_hbm, hsx_hbm, u_hbm):
    pltpu.sync_copy(ss_hbm, ss_t)
    pltpu.sync_copy(sd_hbm, sd_t)
    pltpu.sync_copy(src2d.at[pl.ds(sid * IDXR, IDXR)], src_w)
    pltpu.sync_copy(dst2d.at[pl.ds(sid * IDXR, IDXR)], dst_w)

    zero16 = jnp.zeros((16,), _f32)

    @pl.loop(0, CHUNK)
    def _(r):
      for q in range(ROW // 16):
        stag0[r, pl.ds(q * 16, 16)] = zero16

    base_r = sid * RPW
    for z in range(RPW // CHUNK):
      pltpu.sync_copy(stag0, u_sp.at[pl.ds(base_r + z * CHUNK, CHUNK)])
    if RPW % CHUNK:
      pltpu.sync_copy(stag0.at[pl.ds(0, RPW % CHUNK)],
                      u_sp.at[pl.ds(base_r + (RPW // CHUNK) * CHUNK,
                                    RPW % CHUNK)])
    plsc.subcore_barrier()

    unit16 = (lax.iota(jnp.int32, 16) == 0).astype(_f32)

    def g_descs(c, buf, sem):
      return [pltpu.make_async_copy(hsx_hbm.at[src_w.at[c * NSUB + j]],
                                    buf.at[pl.ds(j * SUB, SUB)], sem)
              for j in range(NSUB)]

    def s_descs(c, buf, sem):
      return [pltpu.make_async_copy(buf.at[pl.ds(j * SUB, SUB)],
                                    u_sp.at[dst_w.at[c * NSUB + j]], sem)
              for j in range(NSUB)]

    def fire_gather(c, buf, sem):
      for j in range(NSUB):
        pltpu.async_copy(hsx_hbm.at[src_w.at[c * NSUB + j]],
                         buf.at[pl.ds(j * SUB, SUB)], sem)

    def wait_gather(c, buf, sem):
      for de in g_descs(c, buf, sem):
        de.wait()

    def fire_scatter(c, buf, sem):
      for j in range(NSUB):
        pltpu.async_copy(buf.at[pl.ds(j * SUB, SUB)],
                         u_sp.at[dst_w.at[c * NSUB + j]], sem, add=True)

    def wait_scatter(c, buf, sem):
      for de in s_descs(c, buf, sem):
        de.wait()

    def compute(c, rows, stag):
      for j in range(NSUB):
        @plsc.parallel_loop(0, SUB, step=16, unroll=2)
        def _(k, j=j):
          ri = c * NSUB + j
          s16 = src_w[ri, pl.ds(k, 16)]
          d16 = dst_w[ri, pl.ds(k, 16)]
          e = plsc.load_gather(ss_t, [s16]) + plsc.load_gather(sd_t, [d16])
          e = jnp.maximum(e, e * 0.2)
          ex = jnp.exp(e)
          for t in range(16):
            r = j * SUB + k + t
            exr = ex[t]
            for q in range(H // 16):
              stag[r, pl.ds(q * 16, 16)] = rows[r, pl.ds(q * 16, 16)] * exr
            stag[r, pl.ds(H, 16)] = unit16 * exr

    fire_gather(0, rows0, sg0)

    @pl.loop(0, NCHUNK - 1, step=2)
    def _(cc):
      fire_gather(cc + 1, rows1, sg1)
      wait_gather(cc, rows0, sg0)
      @pl.when(cc > 0)
      def _():
        wait_scatter(cc - 2, stag0, sc0)
      compute(cc, rows0, stag0)
      fire_scatter(cc, stag0, sc0)
      fire_gather(cc + 2, rows0, sg0)
      wait_gather(cc + 1, rows1, sg1)
      @pl.when(cc > 0)
      def _():
        wait_scatter(cc - 1, stag1, sc1)
      compute(cc + 1, rows1, stag1)
      fire_scatter(cc + 1, stag1, sc1)

    wait_gather(NCHUNK - 1, rows0, sg0)
    wait_scatter(NCHUNK - 3, stag0, sc0)
    compute(NCHUNK - 1, rows0, stag0)
    fire_scatter(NCHUNK - 1, stag0, sc0)
    wait_scatter(NCHUNK - 2, stag1, sc1)
    wait_scatter(NCHUNK - 1, stag0, sc0)

    plsc.subcore_barrier()
    pltpu.sync_copy(u_sp.at[pl.ds(base_r, RPW)], u_hbm.at[pl.ds(base_r, RPW)])

  @pl.when(cid == 0)
  def _():
    run_rel(src_pv, dst_pv, ss_pv, sd_pv, hs_pv, u_pv)

  @pl.when(cid == 1)
  def _():
    run_rel(src_vp, dst_vp, ss_vp, sd_vp, hs_vp, u_vp)


def _edge_pass(src_pv, dst_pv, src_vp, dst_vp,
               ss_pv, sd_pv, ss_vp, sd_vp, hs_pv, hs_vp):
  mesh = plsc.VectorSubcoreMesh(core_axis_name="c", subcore_axis_name="s")
  kern = pl.kernel(
      _edge_body,
      out_type=(jax.ShapeDtypeStruct((N, ROW), _f32),
                jax.ShapeDtypeStruct((N, ROW), _f32)),
      mesh=mesh,
      compiler_params=pltpu.CompilerParams(use_tc_tiling_on_sc=False,
                                           needs_layout_passes=False),
      scratch_types=[
          pltpu.VMEM((N,), _f32),
          pltpu.VMEM((N,), _f32),
          pltpu.VMEM((IDXR, SUB), jnp.int32),
          pltpu.VMEM((IDXR, SUB), jnp.int32),
          pltpu.VMEM((CHUNK, H), _f32),
          pltpu.VMEM((CHUNK, H), _f32),
          pltpu.VMEM((CHUNK, ROW), _f32),
          pltpu.VMEM((CHUNK, ROW), _f32),
          pltpu.VMEM_SHARED((N, ROW), _f32),
          pltpu.SemaphoreType.DMA,
          pltpu.SemaphoreType.DMA,
          pltpu.SemaphoreType.DMA,
          pltpu.SemaphoreType.DMA,
      ],
  )
  return kern(src_pv, dst_pv, src_vp, dst_vp,
              ss_pv, sd_pv, ss_vp, sd_vp, hs_pv, hs_vp)


def kernel(x_patient, x_visit, edge_index_pv, edge_index_vp,
           l1_pv_Wsrc, l1_pv_Wdst, l1_pv_asrc, l1_pv_adst, l1_pv_b,
           l1_vp_Wsrc, l1_vp_Wdst, l1_vp_asrc, l1_vp_adst, l1_vp_b,
           l2_pv_Wsrc, l2_pv_Wdst, l2_pv_asrc, l2_pv_adst, l2_pv_b,
           l2_vp_Wsrc, l2_vp_Wdst, l2_vp_asrc, l2_vp_adst, l2_vp_b,
           head_p_W, head_p_b, head_v_W, head_v_b):
  src_pv = edge_index_pv[0].reshape(E // SUB, SUB)
  dst_pv = edge_index_pv[1].reshape(E // SUB, SUB)
  src_vp = edge_index_vp[0].reshape(E // SUB, SUB)
  dst_vp = edge_index_vp[1].reshape(E // SUB, SUB)

  hs1_pv, ss1_pv, sd1_pv, hs1_vp, ss1_vp, sd1_vp = _dense_pair(
      x_patient, x_visit,
      l1_pv_Wsrc, l1_pv_asrc, l1_pv_Wdst, l1_pv_adst,
      l1_vp_Wsrc, l1_vp_asrc, l1_vp_Wdst, l1_vp_adst)

  u1_pv, u1_vp = _edge_pass(
      src_pv, dst_pv, src_vp, dst_vp,
      ss1_pv, sd1_pv, ss1_vp, sd1_vp, hs1_pv, hs1_vp)

  hs2_pv, ss2_pv, sd2_pv, hs2_vp, ss2_vp, sd2_vp = _combine_dense(
      u1_pv, u1_vp, l1_pv_b, l1_vp_b,
      l2_pv_Wsrc, l2_pv_asrc, l2_pv_Wdst, l2_pv_adst,
      l2_vp_Wsrc, l2_vp_asrc, l2_vp_Wdst, l2_vp_adst)

  u2_pv, u2_vp = _edge_pass(
      src_pv, dst_pv, src_vp, dst_vp,
      ss2_pv, sd2_pv, ss2_vp, sd2_vp, hs2_pv, hs2_vp)

  out_p, out_v = _final(u2_pv, u2_vp, l2_pv_b, l2_vp_b,
                        head_p_W, head_p_b, head_v_W, head_v_b)
  return (out_p, out_v)


# speedup vs baseline: 61.6673x; 1.1562x over previous
import functools

import jax
import jax.numpy as jnp
from jax import lax
from jax.experimental import pallas as pl
from jax.experimental.pallas import tpu as pltpu
from jax.experimental.pallas import tpu_sc as plsc

N = 10000
E = 160000
D = 128
H = 64
OUT = 32
ROW = 80
SUB = 80
NSUB = 1
CHUNK = SUB * NSUB
NW = 16
EPW = E // NW
NCHUNK = EPW // CHUNK
IDXR = EPW // SUB
RPW = N // NW

_f32 = jnp.float32


def _dot(a, b):
  return jnp.dot(a.astype(jnp.bfloat16), b.astype(jnp.bfloat16),
                 preferred_element_type=_f32)


def _dense_pair_core(xa, xb, ws_ab, as_ab, wd_ab, ad_ab,
                     ws_ba, as_ba, wd_ba, ad_ba,
                     hs_ab_ref, ss_ab_ref, sd_ab_ref,
                     hs_ba_ref, ss_ba_ref, sd_ba_ref):
  hs_ab = _dot(xa, ws_ab)
  hs_ab_ref[...] = hs_ab
  ss_ab_ref[...] = jnp.sum(hs_ab * as_ab, axis=1)
  sd_ab_ref[...] = jnp.sum(_dot(xb, wd_ab) * ad_ab, axis=1)
  hs_ba = _dot(xb, ws_ba)
  hs_ba_ref[...] = hs_ba
  ss_ba_ref[...] = jnp.sum(hs_ba * as_ba, axis=1)
  sd_ba_ref[...] = jnp.sum(_dot(xa, wd_ba) * ad_ba, axis=1)


def _dense_body(xa_ref, xb_ref,
                ws_ab_ref, as_ab_ref, wd_ab_ref, ad_ab_ref,
                ws_ba_ref, as_ba_ref, wd_ba_ref, ad_ba_ref,
                *out_refs):
  _dense_pair_core(xa_ref[...], xb_ref[...],
                   ws_ab_ref[...], as_ab_ref[...], wd_ab_ref[...],
                   ad_ab_ref[...], ws_ba_ref[...], as_ba_ref[...],
                   wd_ba_ref[...], ad_ba_ref[...], *out_refs)


_PAIR_OUT = (
    jax.ShapeDtypeStruct((N, H), _f32),
    jax.ShapeDtypeStruct((N,), _f32),
    jax.ShapeDtypeStruct((N,), _f32),
    jax.ShapeDtypeStruct((N, H), _f32),
    jax.ShapeDtypeStruct((N,), _f32),
    jax.ShapeDtypeStruct((N,), _f32),
)


def _dense_pair(xa, xb, ws_ab, aas_ab, wd_ab, ad_ab, ws_ba, aas_ba, wd_ba, ad_ba):
  return pl.pallas_call(_dense_body, out_shape=_PAIR_OUT)(
      xa, xb, ws_ab, aas_ab.reshape(1, H), wd_ab, ad_ab.reshape(1, H),
      ws_ba, aas_ba.reshape(1, H), wd_ba, ad_ba.reshape(1, H))


def _combine(u):
  return u[:, :H] / (u[:, H:H + 1] + 1e-16)


def _combine_dense_body(u_pv_ref, u_vp_ref, b_pv_ref, b_vp_ref,
                        ws_ab_ref, as_ab_ref, wd_ab_ref, ad_ab_ref,
                        ws_ba_ref, as_ba_ref, wd_ba_ref, ad_ba_ref,
                        *out_refs):
  h_v = jnp.maximum(_combine(u_pv_ref[...]) + b_pv_ref[...], 0.0)
  h_p = jnp.maximum(_combine(u_vp_ref[...]) + b_vp_ref[...], 0.0)
  _dense_pair_core(h_p, h_v,
                   ws_ab_ref[...], as_ab_ref[...], wd_ab_ref[...],
                   ad_ab_ref[...], ws_ba_ref[...], as_ba_ref[...],
                   wd_ba_ref[...], ad_ba_ref[...], *out_refs)


def _combine_dense(u_pv, u_vp, b_pv, b_vp,
                   ws_ab, aas_ab, wd_ab, ad_ab, ws_ba, aas_ba, wd_ba, ad_ba):
  return pl.pallas_call(_combine_dense_body, out_shape=_PAIR_OUT)(
      u_pv, u_vp, b_pv.reshape(1, H), b_vp.reshape(1, H),
      ws_ab, aas_ab.reshape(1, H), wd_ab, ad_ab.reshape(1, H),
      ws_ba, aas_ba.reshape(1, H), wd_ba, ad_ba.reshape(1, H))


def _final_body(u_pv_ref, u_vp_ref, b_pv_ref, b_vp_ref,
                hw_p_ref, hb_p_ref, hw_v_ref, hb_v_ref,
                out_p_ref, out_v_ref):
  h_v2 = jnp.maximum(_combine(u_pv_ref[...]) + b_pv_ref[...], 0.0)
  h_p2 = jnp.maximum(_combine(u_vp_ref[...]) + b_vp_ref[...], 0.0)
  out_p_ref[...] = _dot(h_p2, hw_p_ref[...]) + hb_p_ref[...]
  out_v_ref[...] = _dot(h_v2, hw_v_ref[...]) + hb_v_ref[...]


def _final(u_pv, u_vp, b_pv, b_vp, head_p_W, head_p_b, head_v_W, head_v_b):
  out_shape = (
      jax.ShapeDtypeStruct((N, OUT), _f32),
      jax.ShapeDtypeStruct((N, H), _f32),
  )
  return pl.pallas_call(_final_body, out_shape=out_shape)(
      u_pv, u_vp, b_pv.reshape(1, H), b_vp.reshape(1, H),
      head_p_W, head_p_b.reshape(1, OUT), head_v_W, head_v_b.reshape(1, H))


def _edge_body(src_pv, dst_pv, src_vp, dst_vp,
               ss_pv, sd_pv, ss_vp, sd_vp, hs_pv, hs_vp,
               u_pv, u_vp,
               ss_t, sd_t, src_w, dst_w, rows0, rows1, stag0, stag1, u_sp,
               sg0, sg1, sc0, sc1):
  cid = lax.axis_index("c")
  sid = lax.axis_index("s")

  def run_rel(src2d, dst2d, ss_hbm, sd_hbm, hsx_hbm, u_hbm):
    pltpu.sync_copy(ss_hbm, ss_t)
    pltpu.sync_copy(sd_hbm, sd_t)
    pltpu.sync_copy(src2d.at[pl.ds(sid * IDXR, IDXR)], src_w)
    pltpu.sync_copy(dst2d.at[pl.ds(sid * IDXR, IDXR)], dst_w)

    zero16 = jnp.zeros((16,), _f32)

    @plsc.parallel_loop(0, CHUNK, unroll=4)
    def _(r):
      for q in range(ROW // 16):
        stag0[r, pl.ds(q * 16, 16)] = zero16

    base_r = sid * RPW
    for z in range(RPW // CHUNK):
      pltpu.sync_copy(stag0, u_sp.at[pl.ds(base_r + z * CHUNK, CHUNK)])
    if RPW % CHUNK:
      pltpu.sync_copy(stag0.at[pl.ds(0, RPW % CHUNK)],
                      u_sp.at[pl.ds(base_r + (RPW // CHUNK) * CHUNK,
                                    RPW % CHUNK)])
    plsc.subcore_barrier()

    unit16 = (lax.iota(jnp.int32, 16) == 0).astype(_f32)

    def g_descs(c, buf, sem):
      return [pltpu.make_async_copy(hsx_hbm.at[src_w.at[c * NSUB + j]],
                                    buf.at[pl.ds(j * SUB, SUB)], sem)
              for j in range(NSUB)]

    def s_descs(c, buf, sem):
      return [pltpu.make_async_copy(buf.at[pl.ds(j * SUB, SUB)],
                                    u_sp.at[dst_w.at[c * NSUB + j]], sem)
              for j in range(NSUB)]

    def fire_gather(c, buf, sem):
      for j in range(NSUB):
        pltpu.async_copy(hsx_hbm.at[src_w.at[c * NSUB + j]],
                         buf.at[pl.ds(j * SUB, SUB)], sem)

    def wait_gather(c, buf, sem):
      for de in g_descs(c, buf, sem):
        de.wait()

    def fire_scatter(c, buf, sem):
      for j in range(NSUB):
        pltpu.async_copy(buf.at[pl.ds(j * SUB, SUB)],
                         u_sp.at[dst_w.at[c * NSUB + j]], sem, add=True)

    def wait_scatter(c, buf, sem):
      for de in s_descs(c, buf, sem):
        de.wait()

    def compute(c, rows, stag):
      for j in range(NSUB):
        @plsc.parallel_loop(0, SUB, step=16, unroll=4)
        def _(k, j=j):
          ri = c * NSUB + j
          s16 = src_w[ri, pl.ds(k, 16)]
          d16 = dst_w[ri, pl.ds(k, 16)]
          e = plsc.load_gather(ss_t, [s16]) + plsc.load_gather(sd_t, [d16])
          e = jnp.maximum(e, e * 0.2)
          ex = jnp.exp(e)
          for t in range(16):
            r = j * SUB + k + t
            exr = ex[t]
            for q in range(H // 16):
              stag[r, pl.ds(q * 16, 16)] = rows[r, pl.ds(q * 16, 16)] * exr
            stag[r, pl.ds(H, 16)] = unit16 * exr

    fire_gather(0, rows0, sg0)

    @pl.loop(0, NCHUNK - 1, step=2)
    def _(cc):
      fire_gather(cc + 1, rows1, sg1)
      wait_gather(cc, rows0, sg0)
      @pl.when(cc > 0)
      def _():
        wait_scatter(cc - 2, stag0, sc0)
      compute(cc, rows0, stag0)
      fire_scatter(cc, stag0, sc0)
      fire_gather(cc + 2, rows0, sg0)
      wait_gather(cc + 1, rows1, sg1)
      @pl.when(cc > 0)
      def _():
        wait_scatter(cc - 1, stag1, sc1)
      compute(cc + 1, rows1, stag1)
      fire_scatter(cc + 1, stag1, sc1)

    wait_gather(NCHUNK - 1, rows0, sg0)
    wait_scatter(NCHUNK - 3, stag0, sc0)
    compute(NCHUNK - 1, rows0, stag0)
    fire_scatter(NCHUNK - 1, stag0, sc0)
    wait_scatter(NCHUNK - 2, stag1, sc1)
    wait_scatter(NCHUNK - 1, stag0, sc0)

    plsc.subcore_barrier()
    pltpu.sync_copy(u_sp.at[pl.ds(base_r, RPW)], u_hbm.at[pl.ds(base_r, RPW)])

  @pl.when(cid == 0)
  def _():
    run_rel(src_pv, dst_pv, ss_pv, sd_pv, hs_pv, u_pv)

  @pl.when(cid == 1)
  def _():
    run_rel(src_vp, dst_vp, ss_vp, sd_vp, hs_vp, u_vp)


def _edge_pass(src_pv, dst_pv, src_vp, dst_vp,
               ss_pv, sd_pv, ss_vp, sd_vp, hs_pv, hs_vp):
  mesh = plsc.VectorSubcoreMesh(core_axis_name="c", subcore_axis_name="s")
  kern = pl.kernel(
      _edge_body,
      out_type=(jax.ShapeDtypeStruct((N, ROW), _f32),
                jax.ShapeDtypeStruct((N, ROW), _f32)),
      mesh=mesh,
      compiler_params=pltpu.CompilerParams(use_tc_tiling_on_sc=False,
                                           needs_layout_passes=False),
      scratch_types=[
          pltpu.VMEM((N,), _f32),
          pltpu.VMEM((N,), _f32),
          pltpu.VMEM((IDXR, SUB), jnp.int32),
          pltpu.VMEM((IDXR, SUB), jnp.int32),
          pltpu.VMEM((CHUNK, H), _f32),
          pltpu.VMEM((CHUNK, H), _f32),
          pltpu.VMEM((CHUNK, ROW), _f32),
          pltpu.VMEM((CHUNK, ROW), _f32),
          pltpu.VMEM_SHARED((N, ROW), _f32),
          pltpu.SemaphoreType.DMA,
          pltpu.SemaphoreType.DMA,
          pltpu.SemaphoreType.DMA,
          pltpu.SemaphoreType.DMA,
      ],
  )
  return kern(src_pv, dst_pv, src_vp, dst_vp,
              ss_pv, sd_pv, ss_vp, sd_vp, hs_pv, hs_vp)


def kernel(x_patient, x_visit, edge_index_pv, edge_index_vp,
           l1_pv_Wsrc, l1_pv_Wdst, l1_pv_asrc, l1_pv_adst, l1_pv_b,
           l1_vp_Wsrc, l1_vp_Wdst, l1_vp_asrc, l1_vp_adst, l1_vp_b,
           l2_pv_Wsrc, l2_pv_Wdst, l2_pv_asrc, l2_pv_adst, l2_pv_b,
           l2_vp_Wsrc, l2_vp_Wdst, l2_vp_asrc, l2_vp_adst, l2_vp_b,
           head_p_W, head_p_b, head_v_W, head_v_b):
  src_pv = edge_index_pv[0].reshape(E // SUB, SUB)
  dst_pv = edge_index_pv[1].reshape(E // SUB, SUB)
  src_vp = edge_index_vp[0].reshape(E // SUB, SUB)
  dst_vp = edge_index_vp[1].reshape(E // SUB, SUB)

  hs1_pv, ss1_pv, sd1_pv, hs1_vp, ss1_vp, sd1_vp = _dense_pair(
      x_patient, x_visit,
      l1_pv_Wsrc, l1_pv_asrc, l1_pv_Wdst, l1_pv_adst,
      l1_vp_Wsrc, l1_vp_asrc, l1_vp_Wdst, l1_vp_adst)

  u1_pv, u1_vp = _edge_pass(
      src_pv, dst_pv, src_vp, dst_vp,
      ss1_pv, sd1_pv, ss1_vp, sd1_vp, hs1_pv, hs1_vp)

  hs2_pv, ss2_pv, sd2_pv, hs2_vp, ss2_vp, sd2_vp = _combine_dense(
      u1_pv, u1_vp, l1_pv_b, l1_vp_b,
      l2_pv_Wsrc, l2_pv_asrc, l2_pv_Wdst, l2_pv_adst,
      l2_vp_Wsrc, l2_vp_asrc, l2_vp_Wdst, l2_vp_adst)

  u2_pv, u2_vp = _edge_pass(
      src_pv, dst_pv, src_vp, dst_vp,
      ss2_pv, sd2_pv, ss2_vp, sd2_vp, hs2_pv, hs2_vp)

  out_p, out_v = _final(u2_pv, u2_vp, l2_pv_b, l2_vp_b,
                        head_p_W, head_p_b, head_v_W, head_v_b)
  return (out_p, out_v)
